# Initial kernel scaffold; baseline (speedup 1.0000x reference)
import functools

import jax
import jax.numpy as jnp
from jax import lax
from jax.experimental import pallas as pl
from jax.experimental.pallas import tpu as pltpu
from jax.experimental.pallas import tpu_sc as plsc

NI = 4000
NU = 6000
NN = NI + NU
NR = 5
NBASE = 8
D = 128
B = 16
L = 512
E = 320000

NC = 2
NS = 16
EP = 327680
ET = EP // (NC * NS)
ECT = EP // NS
CH = 128
NCHUNK = ET // CH
NVEC = ET // 16

SEG_ROWS = 240
C2_ROWS = 32
UP = 6016
IP = 4096

PAD_SRC = 4095
PAD_DST = NI + 6004
PAD_TY = 0

_SC_PARAMS = pltpu.CompilerParams(needs_layout_passes=False)
_MESH = plsc.VectorSubcoreMesh(core_axis_name="c", subcore_axis_name="s")


def _zero16():
    return jnp.zeros((16,), jnp.float32)


def _ones16():
    return jnp.ones((16,), jnp.float32)


def _w2_body(comp_ref, basis_ref, out_ref):
    for r in range(NR):
        acc = comp_ref[r, 0] * basis_ref[0]
        for b in range(1, NBASE):
            acc += comp_ref[r, b] * basis_ref[b]
        out_ref[r] = acc


def _make_w2(comp, basis):
    out = pl.pallas_call(
        _w2_body,
        grid=(4,),
        in_specs=[
            pl.BlockSpec(memory_space=pltpu.SMEM),
            pl.BlockSpec((NBASE, 1000, D), lambda j: (0, j, 0)),
        ],
        out_specs=pl.BlockSpec((NR, 1000, D), lambda j: (0, j, 0)),
        out_shape=jax.ShapeDtypeStruct((NR, NI, D), jnp.float32),
    )(comp, basis)
    return out.reshape(NR * NI, D)


@functools.partial(
    pl.kernel,
    out_type=(
        jax.ShapeDtypeStruct((NC, UP, D), jnp.float32),
        jax.ShapeDtypeStruct((C2_ROWS, D), jnp.float32),
    ),
    mesh=_MESH,
    compiler_params=_SC_PARAMS,
    scratch_types=[
        pltpu.VMEM((ET,), jnp.int32),
        pltpu.VMEM((ET,), jnp.int32),
        pltpu.VMEM((ET,), jnp.int32),
        pltpu.VMEM((ET,), jnp.float32),
        pltpu.VMEM((NCHUNK, CH), jnp.int32),
        pltpu.VMEM((CH,), jnp.int32),
        pltpu.VMEM((SEG_ROWS, D), jnp.float32),
        pltpu.VMEM((C2_ROWS, D), jnp.float32),
        pltpu.VMEM((CH, D), jnp.float32),
        pltpu.VMEM((CH, D), jnp.float32),
        pltpu.VMEM((16,), jnp.int32),
        pltpu.VMEM_SHARED((UP, D), jnp.float32),
        pltpu.VMEM_SHARED((SEG_ROWS, D), jnp.float32),
        pltpu.VMEM_SHARED((C2_ROWS, D), jnp.float32),
        pltpu.SemaphoreType.DMA,
    ],
)
def _sc_stage_a(es_hbm, ed_hbm, et_hbm, w2_hbm, accp_hbm, cnt2_hbm,
                ib_ty, ib_sr, ib_ds, wb, uix, uix1, c1l, c2l, gb, zb, ixr,
                ush, c1sh, c2sh, sem):
    cid = lax.axis_index("c")
    sid = lax.axis_index("s")

    @pl.loop(0, CH)
    def _(i):
        for j in range(8):
            zb[i, pl.ds(j * 16, 16)] = _zero16()

    @pl.loop(0, SEG_ROWS)
    def _(i):
        for j in range(8):
            c1l[i, pl.ds(j * 16, 16)] = _zero16()

    @pl.loop(0, C2_ROWS)
    def _(i):
        for j in range(8):
            c2l[i, pl.ds(j * 16, 16)] = _zero16()

    u0 = sid * (UP // NS)
    pltpu.sync_copy(zb, ush.at[pl.ds(u0, CH)])
    pltpu.sync_copy(zb, ush.at[pl.ds(u0 + CH, CH)])
    pltpu.sync_copy(zb.at[pl.ds(0, UP // NS - 2 * CH)],
                    ush.at[pl.ds(u0 + 2 * CH, UP // NS - 2 * CH)])

    @pl.when(sid == 0)
    def _():
        pltpu.sync_copy(zb, c1sh.at[pl.ds(0, CH)])
        pltpu.sync_copy(zb.at[pl.ds(0, SEG_ROWS - CH)],
                        c1sh.at[pl.ds(CH, SEG_ROWS - CH)])

    @pl.when(sid == 1)
    def _():
        pltpu.sync_copy(zb.at[pl.ds(0, C2_ROWS)], c2sh)

    plsc.subcore_barrier()

    for h in range(2):
        base = sid * ECT + h * ET
        pltpu.sync_copy(et_hbm.at[pl.ds(base, ET)], ib_ty)
        pltpu.sync_copy(ed_hbm.at[pl.ds(base, ET)], ib_ds)

        @pl.loop(0, NVEC)
        def _(i):
            ty = ib_ty[pl.ds(i * 16, 16)]
            dv = ib_ds[pl.ds(i * 16, 16)]
            seg = (dv - NI) * NR + ty
            plsc.addupdate_scatter(
                c1l,
                [lax.shift_right_logical(seg, 7), jnp.bitwise_and(seg, 127)],
                _ones16())

        pltpu.sync_copy(es_hbm.at[pl.ds(base, ET)], ib_sr)

        @pl.loop(0, NVEC)
        def _(i):
            sr = ib_sr[pl.ds(i * 16, 16)]
            plsc.addupdate_scatter(
                c2l,
                [lax.shift_right_logical(sr, 7), jnp.bitwise_and(sr, 127)],
                _ones16())

    for k in range(SEG_ROWS // 16):
        ixr[...] = lax.iota(jnp.int32, 16) + 16 * k
        pltpu.sync_copy(c1l.at[pl.ds(16 * k, 16)], c1sh.at[ixr], add=True)
    for k in range(C2_ROWS // 16):
        ixr[...] = lax.iota(jnp.int32, 16) + 16 * k
        pltpu.sync_copy(c2l.at[pl.ds(16 * k, 16)], c2sh.at[ixr], add=True)
    plsc.subcore_barrier()
    pltpu.sync_copy(c1sh, c1l)

    @pl.when(jnp.logical_and(cid == 0, sid == 0))
    def _():
        pltpu.sync_copy(c2sh, c2l)
        pltpu.sync_copy(c2l, cnt2_hbm)

    mbase = cid * (EP // NC) + sid * ET
    pltpu.sync_copy(et_hbm.at[pl.ds(mbase, ET)], ib_ty)
    pltpu.sync_copy(es_hbm.at[pl.ds(mbase, ET)], ib_sr)
    pltpu.sync_copy(ed_hbm.at[pl.ds(mbase, ET)], ib_ds)

    @pl.loop(0, NVEC)
    def _(i):
        ty = ib_ty[pl.ds(i * 16, 16)]
        sr = ib_sr[pl.ds(i * 16, 16)]
        dv = ib_ds[pl.ds(i * 16, 16)]
        seg = (dv - NI) * NR + ty
        cnt = plsc.load_gather(
            c1l,
            [lax.shift_right_logical(seg, 7), jnp.bitwise_and(seg, 127)])
        wb[pl.ds(i * 16, 16)] = 1.0 / cnt
        ib_ty[pl.ds(i * 16, 16)] = ty * NI + sr
        uix[lax.shift_right_logical(i, 3),
            pl.ds(jnp.bitwise_and(i, 7) * 16, 16)] = dv - NI

    @pl.loop(0, NCHUNK)
    def _(j):
        pltpu.async_copy(
            w2_hbm.at[ib_ty.at[pl.ds(j * CH, CH)]], gb, sem).wait()

        @pl.loop(0, CH)
        def _(e):
            w = plsc.load_gather(
                wb, [jnp.full((16,), 1, jnp.int32) * (j * CH + e)])
            for k2 in range(8):
                val = gb[e, pl.ds(k2 * 16, 16)]
                gb[e, pl.ds(k2 * 16, 16)] = val * w

        for k2 in range(8):
            uix1[pl.ds(k2 * 16, 16)] = uix[j, pl.ds(k2 * 16, 16)]
        pltpu.sync_copy(gb, ush.at[uix1], add=True)

    plsc.subcore_barrier()

    pltpu.sync_copy(ush.at[pl.ds(u0, CH)], gb)
    pltpu.sync_copy(gb, accp_hbm.at[cid, pl.ds(u0, CH)])
    pltpu.sync_copy(ush.at[pl.ds(u0 + CH, CH)], gb)
    pltpu.sync_copy(gb, accp_hbm.at[cid, pl.ds(u0 + CH, CH)])
    rem = UP // NS - 2 * CH
    pltpu.sync_copy(ush.at[pl.ds(u0 + 2 * CH, rem)], gb.at[pl.ds(0, rem)])
    pltpu.sync_copy(gb.at[pl.ds(0, rem)],
                    accp_hbm.at[cid, pl.ds(u0 + 2 * CH, rem)])


def _user_emb_body(p_ref, root_ref, bias_ref, out_ref):
    out_ref[...] = p_ref[0] + p_ref[1] + root_ref[...] + bias_ref[...]


def _make_user_emb(accp, root_u, bias_row):
    return pl.pallas_call(
        _user_emb_body,
        out_shape=jax.ShapeDtypeStruct((UP, D), jnp.float32),
    )(accp, root_u, bias_row)


@functools.partial(
    pl.kernel,
    out_type=jax.ShapeDtypeStruct((NC, IP, D), jnp.float32),
    mesh=_MESH,
    compiler_params=_SC_PARAMS,
    scratch_types=[
        pltpu.VMEM((ET,), jnp.int32),
        pltpu.VMEM((ET,), jnp.int32),
        pltpu.VMEM((NCHUNK, CH), jnp.int32),
        pltpu.VMEM((CH,), jnp.int32),
        pltpu.VMEM((CH, D), jnp.float32),
        pltpu.VMEM((CH, D), jnp.float32),
        pltpu.VMEM_SHARED((IP, D), jnp.float32),
        pltpu.SemaphoreType.DMA,
    ],
)
def _sc_stage_b(es_hbm, ed_hbm, ue_hbm, qp_hbm,
                ib_ds, ib_sr, six, six1, gb, zb, iash, sem):
    cid = lax.axis_index("c")
    sid = lax.axis_index("s")

    @pl.loop(0, CH)
    def _(i):
        for j in range(8):
            zb[i, pl.ds(j * 16, 16)] = _zero16()

    i0 = sid * (IP // NS)
    pltpu.sync_copy(zb, iash.at[pl.ds(i0, CH)])
    pltpu.sync_copy(zb, iash.at[pl.ds(i0 + CH, CH)])
    plsc.subcore_barrier()

    mbase = cid * (EP // NC) + sid * ET
    pltpu.sync_copy(ed_hbm.at[pl.ds(mbase, ET)], ib_ds)
    pltpu.sync_copy(es_hbm.at[pl.ds(mbase, ET)], ib_sr)

    @pl.loop(0, NVEC)
    def _(i):
        dv = ib_ds[pl.ds(i * 16, 16)]
        ib_ds[pl.ds(i * 16, 16)] = dv - NI
        six[lax.shift_right_logical(i, 3),
            pl.ds(jnp.bitwise_and(i, 7) * 16, 16)] = ib_sr[pl.ds(i * 16, 16)]

    @pl.loop(0, NCHUNK)
    def _(j):
        pltpu.async_copy(
            ue_hbm.at[ib_ds.at[pl.ds(j * CH, CH)]], gb, sem).wait()
        for k2 in range(8):
            six1[pl.ds(k2 * 16, 16)] = six[j, pl.ds(k2 * 16, 16)]
        pltpu.sync_copy(gb, iash.at[six1], add=True)

    plsc.subcore_barrier()

    pltpu.sync_copy(iash.at[pl.ds(i0, CH)], gb)
    pltpu.sync_copy(gb, qp_hbm.at[cid, pl.ds(i0, CH)])
    pltpu.sync_copy(iash.at[pl.ds(i0 + CH, CH)], gb)
    pltpu.sync_copy(gb, qp_hbm.at[cid, pl.ds(i0 + CH, CH)])


def _item_mean_body(q_ref, c_ref, out_ref):
    out_ref[...] = (q_ref[0] + q_ref[1]) / jnp.maximum(c_ref[...], 1.0)


def _make_item_mean(qp, cnt_col):
    return pl.pallas_call(
        _item_mean_body,
        out_shape=jax.ShapeDtypeStruct((IP, D), jnp.float32),
    )(qp, cnt_col)


@functools.partial(
    pl.kernel,
    out_type=jax.ShapeDtypeStruct((B * L, D), jnp.float32),
    mesh=_MESH,
    compiler_params=_SC_PARAMS,
    scratch_types=[
        pltpu.VMEM((B * L // (NC * NS),), jnp.int32),
        pltpu.VMEM((CH, D), jnp.float32),
        pltpu.SemaphoreType.DMA,
    ],
)
def _sc_ctx_gather(im_hbm, ctx_hbm, h_hbm, ib, gb, sem):
    cid = lax.axis_index("c")
    sid = lax.axis_index("s")
    wid = sid * NC + cid
    per = B * L // (NC * NS)
    base = wid * per
    pltpu.sync_copy(ctx_hbm.at[pl.ds(base, per)], ib)
    for k in range(per // CH):
        pltpu.async_copy(im_hbm.at[ib.at[pl.ds(k * CH, CH)]], gb, sem).wait()
        pltpu.sync_copy(gb, h_hbm.at[pl.ds(base + k * CH, CH)])


def _attn_body(h_ref, ctx_ref, a_ref, b_ref, w1_ref, b1_ref, w2_ref, b2_ref,
               social_ref, proj_ref):
    h = h_ref[0]
    t = jnp.tanh(jnp.dot(h, a_ref[...], preferred_element_type=jnp.float32))
    e = jnp.sum(t * b_ref[...], axis=1, keepdims=True)
    valid = ctx_ref[0] >= 0
    e = jnp.where(valid, e, -1e9)
    m = jnp.max(e, axis=0, keepdims=True)
    ex = jnp.exp(e - m)
    s = jnp.sum(ex, axis=0, keepdims=True)
    alpha = ex / s
    social = h * alpha
    social_ref[0] = social
    pooled = jnp.sum(social, axis=0, keepdims=True)
    x = jnp.maximum(
        jnp.dot(pooled, w1_ref[...], preferred_element_type=jnp.float32)
        + b1_ref[...], 0.0)
    proj_ref[...] = jnp.maximum(
        jnp.dot(x, w2_ref[...], preferred_element_type=jnp.float32)
        + b2_ref[...], 0.0)


def _make_attn(h3, ctx3, attn_a, attn_b_row, fc1_w, fc1_b_row, fc2_w,
               fc2_b_row):
    return pl.pallas_call(
        _attn_body,
        grid=(B,),
        in_specs=[
            pl.BlockSpec((1, L, D), lambda i: (i, 0, 0)),
            pl.BlockSpec((1, L, 1), lambda i: (i, 0, 0)),
            pl.BlockSpec((D, D), lambda i: (0, 0)),
            pl.BlockSpec((1, D), lambda i: (0, 0)),
            pl.BlockSpec((D, D), lambda i: (0, 0)),
            pl.BlockSpec((1, D), lambda i: (0, 0)),
            pl.BlockSpec((D, D), lambda i: (0, 0)),
            pl.BlockSpec((1, D), lambda i: (0, 0)),
        ],
        out_specs=[
            pl.BlockSpec((1, L, D), lambda i: (i, 0, 0)),
            pl.BlockSpec((1, D), lambda i: (i, 0)),
        ],
        out_shape=[
            jax.ShapeDtypeStruct((B, L, D), jnp.float32),
            jax.ShapeDtypeStruct((B, D), jnp.float32),
        ],
    )(h3, ctx3, attn_a, attn_b_row, fc1_w, fc1_b_row, fc2_w, fc2_b_row)


def kernel(context_items, edge_src, edge_dst, edge_type, basis, comp, root,
           rgcn_bias, attn_a, attn_b, fc1_w, fc1_b, fc2_w, fc2_b):
    pad = EP - E
    es = jnp.concatenate(
        [edge_src, jnp.full((pad,), PAD_SRC, jnp.int32)])
    ed = jnp.concatenate(
        [edge_dst, jnp.full((pad,), PAD_DST, jnp.int32)])
    et = jnp.concatenate(
        [edge_type, jnp.full((pad,), PAD_TY, jnp.int32)])

    w2 = _make_w2(comp, basis)

    accp, cnt2 = _sc_stage_a(es, ed, et, w2)

    root_u = jnp.concatenate(
        [root[NI:], jnp.zeros((UP - NU, D), jnp.float32)])
    user_emb = _make_user_emb(accp, root_u, rgcn_bias[None, :])

    qp = _sc_stage_b(es, ed, user_emb)

    cnt_col = cnt2.reshape(IP)[:, None]
    item_mean = _make_item_mean(qp, cnt_col)

    ctx_flat = context_items.reshape(B * L)
    h_flat = _sc_ctx_gather(item_mean, ctx_flat)

    h3 = h_flat.reshape(B, L, D)
    ctx3 = context_items.reshape(B, L, 1)
    social_reps, proj = _make_attn(
        h3, ctx3, attn_a, attn_b[None, :], fc1_w, fc1_b[None, :], fc2_w,
        fc2_b[None, :])
    return proj, social_reps


# speedup vs baseline: 6.6977x; 6.6977x over previous
import functools

import jax
import jax.numpy as jnp
from jax import lax
from jax.experimental import pallas as pl
from jax.experimental.pallas import tpu as pltpu
from jax.experimental.pallas import tpu_sc as plsc

NI = 4000
NU = 6000
NR = 5
NBASE = 8
D = 128
B = 16
L = 512
E = 320000

NC = 2
NS = 16
EP = 327680
ET = EP // (NC * NS)
ECT = EP // NS
CH = 128
NCHUNK = ET // CH
CB = 2048
NCCH = ECT // CB

SRC_OFF = 30720
HROWS = 272
C2_ROWS = 32
UP = 6016
IP = 4096

PAD_SRC = 4095
PAD_DST = NI + 6004
PAD_TY = 0

_SC_PARAMS = pltpu.CompilerParams(needs_layout_passes=False)
_MESH = plsc.VectorSubcoreMesh(core_axis_name="c", subcore_axis_name="s")


def _zero16():
    return jnp.zeros((16,), jnp.float32)


def _ones16():
    return jnp.ones((16,), jnp.float32)


def _hist_idx(flat):
    return [lax.shift_right_logical(flat, 7), jnp.bitwise_and(flat, 127)]


def _w2_body(comp_ref, basis_ref, out_ref):
    for r in range(NR):
        acc = comp_ref[r, 0] * basis_ref[0]
        for b in range(1, NBASE):
            acc += comp_ref[r, b] * basis_ref[b]
        out_ref[r] = acc


def _make_w2(comp, basis):
    out = pl.pallas_call(
        _w2_body,
        grid=(4,),
        in_specs=[
            pl.BlockSpec(memory_space=pltpu.SMEM),
            pl.BlockSpec((NBASE, 1000, D), lambda j: (0, j, 0)),
        ],
        out_specs=pl.BlockSpec((NR, 1000, D), lambda j: (0, j, 0)),
        out_shape=jax.ShapeDtypeStruct((NR, NI, D), jnp.float32),
    )(comp, basis)
    return out.reshape(NR * NI, D)


@functools.partial(
    pl.kernel,
    out_type=(
        jax.ShapeDtypeStruct((NC, UP, D), jnp.float32),
        jax.ShapeDtypeStruct((C2_ROWS, D), jnp.float32),
    ),
    mesh=_MESH,
    compiler_params=_SC_PARAMS,
    scratch_types=[
        pltpu.VMEM((CB,), jnp.int32),
        pltpu.VMEM((CB,), jnp.int32),
        pltpu.VMEM((CH,), jnp.int32),
        pltpu.VMEM((CH,), jnp.int32),
        pltpu.VMEM((CH,), jnp.int32),
        pltpu.VMEM((CH,), jnp.int32),
        pltpu.VMEM((CH,), jnp.float32),
        pltpu.VMEM((HROWS, D), jnp.float32),
        pltpu.VMEM((CH, D), jnp.float32),
        pltpu.VMEM((16,), jnp.int32),
        pltpu.VMEM_SHARED((UP, D), jnp.float32),
        pltpu.VMEM_SHARED((HROWS, D), jnp.float32),
        pltpu.SemaphoreType.DMA,
    ],
)
def _sc_stage_a(es_hbm, ed_hbm, et_hbm, w2_hbm, accp_hbm, cnt2_hbm,
                cbufA, cbufB, mty, msr, mds, uix1, mw, c1l, gb, ixr,
                ush, c1sh, sem):
    cid = lax.axis_index("c")
    sid = lax.axis_index("s")

    @pl.loop(0, HROWS)
    def _(i):
        for j in range(8):
            c1l[i, pl.ds(j * 16, 16)] = _zero16()

    @pl.loop(0, CH)
    def _(i):
        for j in range(8):
            gb[i, pl.ds(j * 16, 16)] = _zero16()

    u0 = sid * (UP // NS)
    pltpu.sync_copy(gb, ush.at[pl.ds(u0, CH)])
    pltpu.sync_copy(gb, ush.at[pl.ds(u0 + CH, CH)])
    pltpu.sync_copy(gb.at[pl.ds(0, UP // NS - 2 * CH)],
                    ush.at[pl.ds(u0 + 2 * CH, UP // NS - 2 * CH)])

    @pl.when(sid == 0)
    def _():
        pltpu.sync_copy(gb, c1sh.at[pl.ds(0, CH)])
        pltpu.sync_copy(gb, c1sh.at[pl.ds(CH, CH)])
        pltpu.sync_copy(gb.at[pl.ds(0, HROWS - 2 * CH)],
                        c1sh.at[pl.ds(2 * CH, HROWS - 2 * CH)])

    plsc.subcore_barrier()

    for h in range(NCCH):
        base = sid * ECT + h * CB
        pltpu.sync_copy(et_hbm.at[pl.ds(base, CB)], cbufA)
        pltpu.sync_copy(ed_hbm.at[pl.ds(base, CB)], cbufB)

        @pl.loop(0, CB // 16)
        def _(i):
            ty = cbufA[pl.ds(i * 16, 16)]
            dv = cbufB[pl.ds(i * 16, 16)]
            seg = (dv - NI) * NR + ty
            plsc.addupdate_scatter(c1l, _hist_idx(seg), _ones16())

    for h in range(NCCH):
        base = sid * ECT + h * CB
        pltpu.sync_copy(es_hbm.at[pl.ds(base, CB)], cbufA)

        @pl.loop(0, CB // 16)
        def _(i):
            sr = cbufA[pl.ds(i * 16, 16)] + SRC_OFF
            plsc.addupdate_scatter(c1l, _hist_idx(sr), _ones16())

    for k in range(HROWS // 16):
        ixr[...] = lax.iota(jnp.int32, 16) + 16 * k
        pltpu.sync_copy(c1l.at[pl.ds(16 * k, 16)], c1sh.at[ixr], add=True)
    plsc.subcore_barrier()
    pltpu.sync_copy(c1sh, c1l)

    @pl.when(jnp.logical_and(cid == 0, sid == 0))
    def _():
        pltpu.sync_copy(c1l.at[pl.ds(SRC_OFF // D, C2_ROWS)], cnt2_hbm)

    mbase = cid * (EP // NC) + sid * ET

    @pl.loop(0, NCHUNK)
    def _(j):
        ebase = mbase + j * CH
        pltpu.sync_copy(et_hbm.at[pl.ds(ebase, CH)], mty)
        pltpu.sync_copy(es_hbm.at[pl.ds(ebase, CH)], msr)
        pltpu.sync_copy(ed_hbm.at[pl.ds(ebase, CH)], mds)

        @pl.loop(0, CH // 16)
        def _(i):
            ty = mty[pl.ds(i * 16, 16)]
            sr = msr[pl.ds(i * 16, 16)]
            dv = mds[pl.ds(i * 16, 16)]
            seg = (dv - NI) * NR + ty
            cnt = plsc.load_gather(c1l, _hist_idx(seg))
            mw[pl.ds(i * 16, 16)] = 1.0 / cnt
            mty[pl.ds(i * 16, 16)] = ty * NI + sr
            uix1[pl.ds(i * 16, 16)] = dv - NI

        pltpu.async_copy(w2_hbm.at[mty], gb, sem).wait()

        @pl.loop(0, CH)
        def _(e):
            w = plsc.load_gather(mw, [jnp.full((16,), 1, jnp.int32) * e])
            for k2 in range(8):
                val = gb[e, pl.ds(k2 * 16, 16)]
                gb[e, pl.ds(k2 * 16, 16)] = val * w

        pltpu.sync_copy(gb, ush.at[uix1], add=True)

    plsc.subcore_barrier()

    pltpu.sync_copy(ush.at[pl.ds(u0, CH)], gb)
    pltpu.sync_copy(gb, accp_hbm.at[cid, pl.ds(u0, CH)])
    pltpu.sync_copy(ush.at[pl.ds(u0 + CH, CH)], gb)
    pltpu.sync_copy(gb, accp_hbm.at[cid, pl.ds(u0 + CH, CH)])
    rem = UP // NS - 2 * CH
    pltpu.sync_copy(ush.at[pl.ds(u0 + 2 * CH, rem)], gb.at[pl.ds(0, rem)])
    pltpu.sync_copy(gb.at[pl.ds(0, rem)],
                    accp_hbm.at[cid, pl.ds(u0 + 2 * CH, rem)])


def _user_emb_body(p_ref, root_ref, bias_ref, out_ref):
    out_ref[...] = p_ref[0] + p_ref[1] + root_ref[...] + bias_ref[...]


def _make_user_emb(accp, root_u, bias_row):
    return pl.pallas_call(
        _user_emb_body,
        out_shape=jax.ShapeDtypeStruct((UP, D), jnp.float32),
    )(accp, root_u, bias_row)


@functools.partial(
    pl.kernel,
    out_type=jax.ShapeDtypeStruct((NC, IP, D), jnp.float32),
    mesh=_MESH,
    compiler_params=_SC_PARAMS,
    scratch_types=[
        pltpu.VMEM((CH,), jnp.int32),
        pltpu.VMEM((CH,), jnp.int32),
        pltpu.VMEM((CH, D), jnp.float32),
        pltpu.VMEM_SHARED((IP, D), jnp.float32),
        pltpu.SemaphoreType.DMA,
    ],
)
def _sc_stage_b(es_hbm, ed_hbm, ue_hbm, qp_hbm, mds, six1, gb, iash, sem):
    cid = lax.axis_index("c")
    sid = lax.axis_index("s")

    @pl.loop(0, CH)
    def _(i):
        for j in range(8):
            gb[i, pl.ds(j * 16, 16)] = _zero16()

    i0 = sid * (IP // NS)
    pltpu.sync_copy(gb, iash.at[pl.ds(i0, CH)])
    pltpu.sync_copy(gb, iash.at[pl.ds(i0 + CH, CH)])
    plsc.subcore_barrier()

    mbase = cid * (EP // NC) + sid * ET

    @pl.loop(0, NCHUNK)
    def _(j):
        ebase = mbase + j * CH
        pltpu.sync_copy(ed_hbm.at[pl.ds(ebase, CH)], mds)
        pltpu.sync_copy(es_hbm.at[pl.ds(ebase, CH)], six1)

        @pl.loop(0, CH // 16)
        def _(i):
            mds[pl.ds(i * 16, 16)] = mds[pl.ds(i * 16, 16)] - NI

        pltpu.async_copy(ue_hbm.at[mds], gb, sem).wait()
        pltpu.sync_copy(gb, iash.at[six1], add=True)

    plsc.subcore_barrier()

    pltpu.sync_copy(iash.at[pl.ds(i0, CH)], gb)
    pltpu.sync_copy(gb, qp_hbm.at[cid, pl.ds(i0, CH)])
    pltpu.sync_copy(iash.at[pl.ds(i0 + CH, CH)], gb)
    pltpu.sync_copy(gb, qp_hbm.at[cid, pl.ds(i0 + CH, CH)])


def _item_mean_body(q_ref, c_ref, out_ref):
    out_ref[...] = (q_ref[0] + q_ref[1]) / jnp.maximum(c_ref[...], 1.0)


def _make_item_mean(qp, cnt_col):
    return pl.pallas_call(
        _item_mean_body,
        out_shape=jax.ShapeDtypeStruct((IP, D), jnp.float32),
    )(qp, cnt_col)


@functools.partial(
    pl.kernel,
    out_type=jax.ShapeDtypeStruct((B * L, D), jnp.float32),
    mesh=_MESH,
    compiler_params=_SC_PARAMS,
    scratch_types=[
        pltpu.VMEM((B * L // (NC * NS),), jnp.int32),
        pltpu.VMEM((CH, D), jnp.float32),
        pltpu.SemaphoreType.DMA,
    ],
)
def _sc_ctx_gather(im_hbm, ctx_hbm, h_hbm, ib, gb, sem):
    cid = lax.axis_index("c")
    sid = lax.axis_index("s")
    wid = sid * NC + cid
    per = B * L // (NC * NS)
    base = wid * per
    pltpu.sync_copy(ctx_hbm.at[pl.ds(base, per)], ib)
    for k in range(per // CH):
        pltpu.async_copy(im_hbm.at[ib.at[pl.ds(k * CH, CH)]], gb, sem).wait()
        pltpu.sync_copy(gb, h_hbm.at[pl.ds(base + k * CH, CH)])


def _attn_body(h_ref, ctx_ref, a_ref, b_ref, w1_ref, b1_ref, w2_ref, b2_ref,
               social_ref, proj_ref):
    h = h_ref[0]
    t = jnp.tanh(jnp.dot(h, a_ref[...], preferred_element_type=jnp.float32))
    e = jnp.sum(t * b_ref[...], axis=1, keepdims=True)
    valid = ctx_ref[0] >= 0
    e = jnp.where(valid, e, -1e9)
    m = jnp.max(e, axis=0, keepdims=True)
    ex = jnp.exp(e - m)
    s = jnp.sum(ex, axis=0, keepdims=True)
    alpha = ex / s
    social = h * alpha
    social_ref[0] = social
    pooled = jnp.sum(social, axis=0, keepdims=True)
    x = jnp.maximum(
        jnp.dot(pooled, w1_ref[...], preferred_element_type=jnp.float32)
        + b1_ref[...], 0.0)
    proj_ref[0] = jnp.maximum(
        jnp.dot(x, w2_ref[...], preferred_element_type=jnp.float32)
        + b2_ref[...], 0.0)


def _make_attn(h3, ctx3, attn_a, attn_b_row, fc1_w, fc1_b_row, fc2_w,
               fc2_b_row):
    return pl.pallas_call(
        _attn_body,
        grid=(B,),
        in_specs=[
            pl.BlockSpec((1, L, D), lambda i: (i, 0, 0)),
            pl.BlockSpec((1, L, 1), lambda i: (i, 0, 0)),
            pl.BlockSpec((D, D), lambda i: (0, 0)),
            pl.BlockSpec((1, D), lambda i: (0, 0)),
            pl.BlockSpec((D, D), lambda i: (0, 0)),
            pl.BlockSpec((1, D), lambda i: (0, 0)),
            pl.BlockSpec((D, D), lambda i: (0, 0)),
            pl.BlockSpec((1, D), lambda i: (0, 0)),
        ],
        out_specs=[
            pl.BlockSpec((1, L, D), lambda i: (i, 0, 0)),
            pl.BlockSpec((1, 1, D), lambda i: (i, 0, 0)),
        ],
        out_shape=[
            jax.ShapeDtypeStruct((B, L, D), jnp.float32),
            jax.ShapeDtypeStruct((B, 1, D), jnp.float32),
        ],
    )(h3, ctx3, attn_a, attn_b_row, fc1_w, fc1_b_row, fc2_w, fc2_b_row)


def kernel(context_items, edge_src, edge_dst, edge_type, basis, comp, root,
           rgcn_bias, attn_a, attn_b, fc1_w, fc1_b, fc2_w, fc2_b):
    pad = EP - E
    es = jnp.concatenate(
        [edge_src, jnp.full((pad,), PAD_SRC, jnp.int32)])
    ed = jnp.concatenate(
        [edge_dst, jnp.full((pad,), PAD_DST, jnp.int32)])
    et = jnp.concatenate(
        [edge_type, jnp.full((pad,), PAD_TY, jnp.int32)])

    w2 = _make_w2(comp, basis)

    accp, cnt2 = _sc_stage_a(es, ed, et, w2)

    root_u = jnp.concatenate(
        [root[NI:], jnp.zeros((UP - NU, D), jnp.float32)])
    user_emb = _make_user_emb(accp, root_u, rgcn_bias[None, :])

    qp = _sc_stage_b(es, ed, user_emb)

    cnt_col = cnt2.reshape(IP)[:, None]
    item_mean = _make_item_mean(qp, cnt_col)

    ctx_flat = context_items.reshape(B * L)
    h_flat = _sc_ctx_gather(item_mean, ctx_flat)

    h3 = h_flat.reshape(B, L, D)
    ctx3 = context_items.reshape(B, L, 1)
    social_reps, proj3 = _make_attn(
        h3, ctx3, attn_a, attn_b[None, :], fc1_w, fc1_b[None, :], fc2_w,
        fc2_b[None, :])
    return proj3.reshape(B, D), social_reps


# speedup vs baseline: 8.5573x; 1.2776x over previous
import functools

import jax
import jax.numpy as jnp
from jax import lax
from jax.experimental import pallas as pl
from jax.experimental.pallas import tpu as pltpu
from jax.experimental.pallas import tpu_sc as plsc

NI = 4000
NU = 6000
NR = 5
NBASE = 8
D = 128
B = 16
L = 512
E = 320000

NC = 2
NS = 16
EP = 327680
ET = EP // (NC * NS)
ECT = EP // NS
CH = 128
NCHUNK = ET // CH
CB = 2048
NCCH = ECT // CB
NVEC = ET // 16

SRC_OFF = 30720
HROWS = 272
C2_ROWS = 32
UP = 6016
IP = 4096

PAD_SRC = 4095
PAD_DST = NI + 6004
PAD_TY = 0

_SC_PARAMS = pltpu.CompilerParams(needs_layout_passes=False)
_MESH = plsc.VectorSubcoreMesh(core_axis_name="c", subcore_axis_name="s")


def _zero16():
    return jnp.zeros((16,), jnp.float32)


def _ones16():
    return jnp.ones((16,), jnp.float32)


def _hist_idx(flat):
    return [lax.shift_right_logical(flat, 7), jnp.bitwise_and(flat, 127)]


def _w2_body(comp_ref, basis_ref, out_ref):
    for r in range(NR):
        acc = comp_ref[r, 0] * basis_ref[0]
        for b in range(1, NBASE):
            acc += comp_ref[r, b] * basis_ref[b]
        out_ref[r] = acc


def _make_w2(comp, basis):
    out = pl.pallas_call(
        _w2_body,
        grid=(4,),
        in_specs=[
            pl.BlockSpec(memory_space=pltpu.SMEM),
            pl.BlockSpec((NBASE, 1000, D), lambda j: (0, j, 0)),
        ],
        out_specs=pl.BlockSpec((NR, 1000, D), lambda j: (0, j, 0)),
        out_shape=jax.ShapeDtypeStruct((NR, NI, D), jnp.float32),
    )(comp, basis)
    return out.reshape(NR * NI, D)


@functools.partial(
    pl.kernel,
    out_type=(
        jax.ShapeDtypeStruct((NC, UP, D), jnp.float32),
        jax.ShapeDtypeStruct((C2_ROWS, D), jnp.float32),
    ),
    mesh=_MESH,
    compiler_params=_SC_PARAMS,
    scratch_types=[
        pltpu.VMEM((CB,), jnp.int32),
        pltpu.VMEM((CB,), jnp.int32),
        pltpu.VMEM((CH,), jnp.int32),
        pltpu.VMEM((CH,), jnp.int32),
        pltpu.VMEM((CH,), jnp.int32),
        pltpu.VMEM((CH,), jnp.int32),
        pltpu.VMEM((CH,), jnp.int32),
        pltpu.VMEM((CH,), jnp.int32),
        pltpu.VMEM((CH,), jnp.int32),
        pltpu.VMEM((CH,), jnp.int32),
        pltpu.VMEM((CH,), jnp.float32),
        pltpu.VMEM((CH,), jnp.float32),
        pltpu.VMEM((HROWS, D), jnp.float32),
        pltpu.VMEM((CH, D), jnp.float32),
        pltpu.VMEM((CH, D), jnp.float32),
        pltpu.VMEM((16,), jnp.int32),
        pltpu.VMEM_SHARED((UP, D), jnp.float32),
        pltpu.VMEM_SHARED((HROWS, D), jnp.float32),
        pltpu.SemaphoreType.DMA,
        pltpu.SemaphoreType.DMA,
        pltpu.SemaphoreType.DMA,
    ],
)
def _sc_stage_a(es_hbm, ed_hbm, et_hbm, w2_hbm, accp_hbm, cnt2_hbm,
                cbufA, cbufB, mty0, mty1, msr0, msr1, mds0, mds1,
                uix0, uix1, mw0, mw1, c1l, gb0, gb1, ixr,
                ush, c1sh, semi, semg, sems):
    cid = lax.axis_index("c")
    sid = lax.axis_index("s")
    mty = (mty0, mty1)
    msr = (msr0, msr1)
    mds = (mds0, mds1)
    uix = (uix0, uix1)
    mw = (mw0, mw1)
    gb = (gb0, gb1)

    @pl.loop(0, HROWS)
    def _(i):
        for j in range(8):
            c1l[i, pl.ds(j * 16, 16)] = _zero16()

    @pl.loop(0, CH)
    def _(i):
        for j in range(8):
            gb0[i, pl.ds(j * 16, 16)] = _zero16()

    u0 = sid * (UP // NS)
    pltpu.sync_copy(gb0, ush.at[pl.ds(u0, CH)])
    pltpu.sync_copy(gb0, ush.at[pl.ds(u0 + CH, CH)])
    pltpu.sync_copy(gb0.at[pl.ds(0, UP // NS - 2 * CH)],
                    ush.at[pl.ds(u0 + 2 * CH, UP // NS - 2 * CH)])

    @pl.when(sid == 0)
    def _():
        pltpu.sync_copy(gb0, c1sh.at[pl.ds(0, CH)])
        pltpu.sync_copy(gb0, c1sh.at[pl.ds(CH, CH)])
        pltpu.sync_copy(gb0.at[pl.ds(0, HROWS - 2 * CH)],
                        c1sh.at[pl.ds(2 * CH, HROWS - 2 * CH)])

    plsc.subcore_barrier()

    for h in range(NCCH):
        base = sid * ECT + h * CB
        ca = pltpu.async_copy(et_hbm.at[pl.ds(base, CB)], cbufA, semi)
        cb = pltpu.async_copy(ed_hbm.at[pl.ds(base, CB)], cbufB, semi)
        ca.wait()
        cb.wait()

        @pl.loop(0, CB // 16)
        def _(i):
            ty = cbufA[pl.ds(i * 16, 16)]
            dv = cbufB[pl.ds(i * 16, 16)]
            seg = (dv - NI) * NR + ty
            plsc.addupdate_scatter(c1l, _hist_idx(seg), _ones16())

    for h in range(NCCH):
        base = sid * ECT + h * CB
        pltpu.sync_copy(es_hbm.at[pl.ds(base, CB)], cbufA)

        @pl.loop(0, CB // 16)
        def _(i):
            sr = cbufA[pl.ds(i * 16, 16)] + SRC_OFF
            plsc.addupdate_scatter(c1l, _hist_idx(sr), _ones16())

    for k in range(HROWS // 16):
        ixr[...] = lax.iota(jnp.int32, 16) + 16 * k
        pltpu.sync_copy(c1l.at[pl.ds(16 * k, 16)], c1sh.at[ixr], add=True)
    plsc.subcore_barrier()
    pltpu.sync_copy(c1sh, c1l)

    @pl.when(jnp.logical_and(cid == 0, sid == 0))
    def _():
        pltpu.sync_copy(c1l.at[pl.ds(SRC_OFF // D, C2_ROWS)], cnt2_hbm)

    mbase = cid * (EP // NC) + sid * ET

    def _prep(b):
        @pl.loop(0, CH // 16)
        def _(i):
            ty = mty[b][pl.ds(i * 16, 16)]
            sr = msr[b][pl.ds(i * 16, 16)]
            dv = mds[b][pl.ds(i * 16, 16)]
            seg = (dv - NI) * NR + ty
            cnt = plsc.load_gather(c1l, _hist_idx(seg))
            mw[b][pl.ds(i * 16, 16)] = 1.0 / cnt
            mty[b][pl.ds(i * 16, 16)] = ty * NI + sr
            uix[b][pl.ds(i * 16, 16)] = dv - NI

    def _scale(b):
        @pl.loop(0, CH)
        def _(e):
            w = plsc.load_gather(mw[b], [jnp.full((16,), 1, jnp.int32) * e])
            for k2 in range(8):
                val = gb[b][e, pl.ds(k2 * 16, 16)]
                gb[b][e, pl.ds(k2 * 16, 16)] = val * w

    @pl.loop(0, NCHUNK // 2)
    def _(j0):
        copies = []
        for b in range(2):
            ebase = mbase + (j0 * 2 + b) * CH
            copies.append(
                pltpu.async_copy(et_hbm.at[pl.ds(ebase, CH)], mty[b], semi))
            copies.append(
                pltpu.async_copy(es_hbm.at[pl.ds(ebase, CH)], msr[b], semi))
            copies.append(
                pltpu.async_copy(ed_hbm.at[pl.ds(ebase, CH)], mds[b], semi))
        for c in copies:
            c.wait()
        for b in range(2):
            _prep(b)
        g0 = pltpu.async_copy(w2_hbm.at[mty[0]], gb[0], semg)
        g1 = pltpu.async_copy(w2_hbm.at[mty[1]], gb[1], semg)
        g0.wait()
        _scale(0)
        s0 = pltpu.async_copy(gb[0], ush.at[uix[0]], sems, add=True)
        g1.wait()
        _scale(1)
        s1 = pltpu.async_copy(gb[1], ush.at[uix[1]], sems, add=True)
        s0.wait()
        s1.wait()

    plsc.subcore_barrier()

    pltpu.sync_copy(ush.at[pl.ds(u0, CH)], gb0)
    pltpu.sync_copy(gb0, accp_hbm.at[cid, pl.ds(u0, CH)])
    pltpu.sync_copy(ush.at[pl.ds(u0 + CH, CH)], gb0)
    pltpu.sync_copy(gb0, accp_hbm.at[cid, pl.ds(u0 + CH, CH)])
    rem = UP // NS - 2 * CH
    pltpu.sync_copy(ush.at[pl.ds(u0 + 2 * CH, rem)], gb0.at[pl.ds(0, rem)])
    pltpu.sync_copy(gb0.at[pl.ds(0, rem)],
                    accp_hbm.at[cid, pl.ds(u0 + 2 * CH, rem)])


def _user_emb_body(p_ref, root_ref, bias_ref, out_ref):
    out_ref[...] = p_ref[0] + p_ref[1] + root_ref[...] + bias_ref[...]


def _make_user_emb(accp, root_u, bias_row):
    return pl.pallas_call(
        _user_emb_body,
        out_shape=jax.ShapeDtypeStruct((UP, D), jnp.float32),
    )(accp, root_u, bias_row)


@functools.partial(
    pl.kernel,
    out_type=jax.ShapeDtypeStruct((NC, IP, D), jnp.float32),
    mesh=_MESH,
    compiler_params=_SC_PARAMS,
    scratch_types=[
        pltpu.VMEM((ET,), jnp.int32),
        pltpu.VMEM((ET,), jnp.int32),
        pltpu.VMEM((CH,), jnp.int32),
        pltpu.VMEM((CH,), jnp.int32),
        pltpu.VMEM((CH,), jnp.int32),
        pltpu.VMEM((CH,), jnp.int32),
        pltpu.VMEM((CH, D), jnp.float32),
        pltpu.VMEM((CH, D), jnp.float32),
        pltpu.VMEM((CH, D), jnp.float32),
        pltpu.VMEM((CH, D), jnp.float32),
        pltpu.VMEM_SHARED((IP, D), jnp.float32),
        pltpu.SemaphoreType.DMA,
        pltpu.SemaphoreType.DMA,
    ],
)
def _sc_stage_b(es_hbm, ed_hbm, ue_hbm, qp_hbm, ib_ds, ib_sr,
                six0, six1, six2, six3, gb0, gb1, gb2, gb3,
                iash, semg, sems):
    cid = lax.axis_index("c")
    sid = lax.axis_index("s")
    six = (six0, six1, six2, six3)
    gb = (gb0, gb1, gb2, gb3)

    @pl.loop(0, CH)
    def _(i):
        for j in range(8):
            gb0[i, pl.ds(j * 16, 16)] = _zero16()

    i0 = sid * (IP // NS)
    pltpu.sync_copy(gb0, iash.at[pl.ds(i0, CH)])
    pltpu.sync_copy(gb0, iash.at[pl.ds(i0 + CH, CH)])
    plsc.subcore_barrier()

    mbase = cid * (EP // NC) + sid * ET
    cd = pltpu.async_copy(ed_hbm.at[pl.ds(mbase, ET)], ib_ds, semg)
    cs = pltpu.async_copy(es_hbm.at[pl.ds(mbase, ET)], ib_sr, sems)
    cd.wait()
    cs.wait()

    @pl.loop(0, NVEC)
    def _(i):
        ib_ds[pl.ds(i * 16, 16)] = ib_ds[pl.ds(i * 16, 16)] - NI

    @pl.loop(0, NCHUNK // 4)
    def _(j0):
        gs = []
        for b in range(4):
            j = j0 * 4 + b
            gs.append(pltpu.async_copy(
                ue_hbm.at[ib_ds.at[pl.ds(j * CH, CH)]], gb[b], semg))
        for b in range(4):
            j = j0 * 4 + b

            @pl.loop(0, CH // 16)
            def _(i):
                six[b][pl.ds(i * 16, 16)] = ib_sr[pl.ds(j * CH + i * 16, 16)]

        ss = []
        for b in range(4):
            gs[b].wait()
            ss.append(pltpu.async_copy(
                gb[b], iash.at[six[b]], sems, add=True))
        for b in range(4):
            ss[b].wait()

    plsc.subcore_barrier()

    pltpu.sync_copy(iash.at[pl.ds(i0, CH)], gb0)
    pltpu.sync_copy(gb0, qp_hbm.at[cid, pl.ds(i0, CH)])
    pltpu.sync_copy(iash.at[pl.ds(i0 + CH, CH)], gb0)
    pltpu.sync_copy(gb0, qp_hbm.at[cid, pl.ds(i0 + CH, CH)])


def _item_mean_body(q_ref, c_ref, out_ref):
    out_ref[...] = (q_ref[0] + q_ref[1]) / jnp.maximum(c_ref[...], 1.0)


def _make_item_mean(qp, cnt_col):
    return pl.pallas_call(
        _item_mean_body,
        out_shape=jax.ShapeDtypeStruct((IP, D), jnp.float32),
    )(qp, cnt_col)


@functools.partial(
    pl.kernel,
    out_type=jax.ShapeDtypeStruct((B * L, D), jnp.float32),
    mesh=_MESH,
    compiler_params=_SC_PARAMS,
    scratch_types=[
        pltpu.VMEM((B * L // (NC * NS),), jnp.int32),
        pltpu.VMEM((CH, D), jnp.float32),
        pltpu.SemaphoreType.DMA,
    ],
)
def _sc_ctx_gather(im_hbm, ctx_hbm, h_hbm, ib, gb, sem):
    cid = lax.axis_index("c")
    sid = lax.axis_index("s")
    wid = sid * NC + cid
    per = B * L // (NC * NS)
    base = wid * per
    pltpu.sync_copy(ctx_hbm.at[pl.ds(base, per)], ib)
    for k in range(per // CH):
        pltpu.async_copy(im_hbm.at[ib.at[pl.ds(k * CH, CH)]], gb, sem).wait()
        pltpu.sync_copy(gb, h_hbm.at[pl.ds(base + k * CH, CH)])


def _attn_body(h_ref, ctx_ref, a_ref, b_ref, w1_ref, b1_ref, w2_ref, b2_ref,
               social_ref, proj_ref):
    h = h_ref[0]
    t = jnp.tanh(jnp.dot(h, a_ref[...], preferred_element_type=jnp.float32))
    e = jnp.sum(t * b_ref[...], axis=1, keepdims=True)
    valid = ctx_ref[0] >= 0
    e = jnp.where(valid, e, -1e9)
    m = jnp.max(e, axis=0, keepdims=True)
    ex = jnp.exp(e - m)
    s = jnp.sum(ex, axis=0, keepdims=True)
    alpha = ex / s
    social = h * alpha
    social_ref[0] = social
    pooled = jnp.sum(social, axis=0, keepdims=True)
    x = jnp.maximum(
        jnp.dot(pooled, w1_ref[...], preferred_element_type=jnp.float32)
        + b1_ref[...], 0.0)
    proj_ref[0] = jnp.maximum(
        jnp.dot(x, w2_ref[...], preferred_element_type=jnp.float32)
        + b2_ref[...], 0.0)


def _make_attn(h3, ctx3, attn_a, attn_b_row, fc1_w, fc1_b_row, fc2_w,
               fc2_b_row):
    return pl.pallas_call(
        _attn_body,
        grid=(B,),
        in_specs=[
            pl.BlockSpec((1, L, D), lambda i: (i, 0, 0)),
            pl.BlockSpec((1, L, 1), lambda i: (i, 0, 0)),
            pl.BlockSpec((D, D), lambda i: (0, 0)),
            pl.BlockSpec((1, D), lambda i: (0, 0)),
            pl.BlockSpec((D, D), lambda i: (0, 0)),
            pl.BlockSpec((1, D), lambda i: (0, 0)),
            pl.BlockSpec((D, D), lambda i: (0, 0)),
            pl.BlockSpec((1, D), lambda i: (0, 0)),
        ],
        out_specs=[
            pl.BlockSpec((1, L, D), lambda i: (i, 0, 0)),
            pl.BlockSpec((1, 1, D), lambda i: (i, 0, 0)),
        ],
        out_shape=[
            jax.ShapeDtypeStruct((B, L, D), jnp.float32),
            jax.ShapeDtypeStruct((B, 1, D), jnp.float32),
        ],
    )(h3, ctx3, attn_a, attn_b_row, fc1_w, fc1_b_row, fc2_w, fc2_b_row)


def kernel(context_items, edge_src, edge_dst, edge_type, basis, comp, root,
           rgcn_bias, attn_a, attn_b, fc1_w, fc1_b, fc2_w, fc2_b):
    pad = EP - E
    es = jnp.concatenate(
        [edge_src, jnp.full((pad,), PAD_SRC, jnp.int32)])
    ed = jnp.concatenate(
        [edge_dst, jnp.full((pad,), PAD_DST, jnp.int32)])
    et = jnp.concatenate(
        [edge_type, jnp.full((pad,), PAD_TY, jnp.int32)])

    w2 = _make_w2(comp, basis)

    accp, cnt2 = _sc_stage_a(es, ed, et, w2)

    root_u = jnp.concatenate(
        [root[NI:], jnp.zeros((UP - NU, D), jnp.float32)])
    user_emb = _make_user_emb(accp, root_u, rgcn_bias[None, :])

    qp = _sc_stage_b(es, ed, user_emb)

    cnt_col = cnt2.reshape(IP)[:, None]
    item_mean = _make_item_mean(qp, cnt_col)

    ctx_flat = context_items.reshape(B * L)
    h_flat = _sc_ctx_gather(item_mean, ctx_flat)

    h3 = h_flat.reshape(B, L, D)
    ctx3 = context_items.reshape(B, L, 1)
    social_reps, proj3 = _make_attn(
        h3, ctx3, attn_a, attn_b[None, :], fc1_w, fc1_b[None, :], fc2_w,
        fc2_b[None, :])
    return proj3.reshape(B, D), social_reps


# speedup vs baseline: 9.0910x; 1.0624x over previous
import functools

import jax
import jax.numpy as jnp
from jax import lax
from jax.experimental import pallas as pl
from jax.experimental.pallas import tpu as pltpu
from jax.experimental.pallas import tpu_sc as plsc

NI = 4000
NU = 6000
NR = 5
NBASE = 8
D = 128
B = 16
L = 512
E = 320000

NC = 2
NS = 16
EP = 327680
ET = EP // (NC * NS)
ECT = EP // NS
CH = 128
NCHUNK = ET // CH
CB = 2048
NCCH = ECT // CB
NVEC = ET // 16

SRC_OFF = 30720
HROWS = 272
C2_ROWS = 32
UP = 6016
IP = 4096

PAD_SRC = 4095
PAD_DST = NI + 6004
PAD_TY = 0

_SC_PARAMS = pltpu.CompilerParams(needs_layout_passes=False)
_MESH = plsc.VectorSubcoreMesh(core_axis_name="c", subcore_axis_name="s")


def _zero16():
    return jnp.zeros((16,), jnp.float32)


def _ones16():
    return jnp.ones((16,), jnp.float32)


def _hist_idx(flat):
    return [lax.shift_right_logical(flat, 7), jnp.bitwise_and(flat, 127)]


def _w2_body(comp_ref, basis_ref, out_ref):
    for r in range(NR):
        acc = comp_ref[r, 0] * basis_ref[0]
        for b in range(1, NBASE):
            acc += comp_ref[r, b] * basis_ref[b]
        out_ref[r] = acc


def _make_w2(comp, basis):
    out = pl.pallas_call(
        _w2_body,
        grid=(4,),
        in_specs=[
            pl.BlockSpec(memory_space=pltpu.SMEM),
            pl.BlockSpec((NBASE, 1000, D), lambda j: (0, j, 0)),
        ],
        out_specs=pl.BlockSpec((NR, 1000, D), lambda j: (0, j, 0)),
        out_shape=jax.ShapeDtypeStruct((NR, NI, D), jnp.float32),
    )(comp, basis)
    return out.reshape(NR * NI, D)


@functools.partial(
    pl.kernel,
    out_type=(
        jax.ShapeDtypeStruct((NC, UP, D), jnp.float32),
        jax.ShapeDtypeStruct((C2_ROWS, D), jnp.float32),
    ),
    mesh=_MESH,
    compiler_params=_SC_PARAMS,
    scratch_types=[
        pltpu.VMEM((CB,), jnp.int32),
        pltpu.VMEM((CB,), jnp.int32),
        pltpu.VMEM((2 * CH,), jnp.int32),
        pltpu.VMEM((2 * CH,), jnp.int32),
        pltpu.VMEM((2 * CH,), jnp.int32),
        pltpu.VMEM((CH,), jnp.int32),
        pltpu.VMEM((CH,), jnp.int32),
        pltpu.VMEM((2 * CH,), jnp.float32),
        pltpu.VMEM((HROWS, D), jnp.float32),
        pltpu.VMEM((CH, D), jnp.float32),
        pltpu.VMEM((CH, D), jnp.float32),
        pltpu.VMEM((16,), jnp.int32),
        pltpu.VMEM_SHARED((UP, D), jnp.float32),
        pltpu.VMEM_SHARED((HROWS, D), jnp.float32),
        pltpu.SemaphoreType.DMA,
        pltpu.SemaphoreType.DMA,
        pltpu.SemaphoreType.DMA,
    ],
)
def _sc_stage_a(es_hbm, ed_hbm, et_hbm, w2_hbm, accp_hbm, cnt2_hbm,
                cbufA, cbufB, mtyg, msrg, mdsg, uix0, uix1, mwg,
                c1l, gb0, gb1, ixr, ush, c1sh, semi, semg, sems):
    cid = lax.axis_index("c")
    sid = lax.axis_index("s")
    uix = (uix0, uix1)
    gb = (gb0, gb1)

    @pl.loop(0, HROWS)
    def _(i):
        for j in range(8):
            c1l[i, pl.ds(j * 16, 16)] = _zero16()

    @pl.loop(0, CH)
    def _(i):
        for j in range(8):
            gb0[i, pl.ds(j * 16, 16)] = _zero16()

    u0 = sid * (UP // NS)
    pltpu.sync_copy(gb0, ush.at[pl.ds(u0, CH)])
    pltpu.sync_copy(gb0, ush.at[pl.ds(u0 + CH, CH)])
    pltpu.sync_copy(gb0.at[pl.ds(0, UP // NS - 2 * CH)],
                    ush.at[pl.ds(u0 + 2 * CH, UP // NS - 2 * CH)])

    @pl.when(sid == 0)
    def _():
        pltpu.sync_copy(gb0, c1sh.at[pl.ds(0, CH)])
        pltpu.sync_copy(gb0, c1sh.at[pl.ds(CH, CH)])
        pltpu.sync_copy(gb0.at[pl.ds(0, HROWS - 2 * CH)],
                        c1sh.at[pl.ds(2 * CH, HROWS - 2 * CH)])

    plsc.subcore_barrier()

    for h in range(NCCH):
        base = sid * ECT + h * CB
        ca = pltpu.async_copy(et_hbm.at[pl.ds(base, CB)], cbufA, semi)
        cb = pltpu.async_copy(ed_hbm.at[pl.ds(base, CB)], cbufB, semi)
        ca.wait()
        cb.wait()

        @pl.loop(0, CB // 16)
        def _(i):
            ty = cbufA[pl.ds(i * 16, 16)]
            dv = cbufB[pl.ds(i * 16, 16)]
            seg = (dv - NI) * NR + ty
            plsc.addupdate_scatter(c1l, _hist_idx(seg), _ones16())

    for h in range(NCCH):
        base = sid * ECT + h * CB
        pltpu.sync_copy(es_hbm.at[pl.ds(base, CB)], cbufA)

        @pl.loop(0, CB // 16)
        def _(i):
            sr = cbufA[pl.ds(i * 16, 16)] + SRC_OFF
            plsc.addupdate_scatter(c1l, _hist_idx(sr), _ones16())

    for k in range(HROWS // 16):
        ixr[...] = lax.iota(jnp.int32, 16) + 16 * k
        pltpu.sync_copy(c1l.at[pl.ds(16 * k, 16)], c1sh.at[ixr], add=True)
    plsc.subcore_barrier()
    pltpu.sync_copy(c1sh, c1l)

    @pl.when(jnp.logical_and(cid == 0, sid == 0))
    def _():
        pltpu.sync_copy(c1l.at[pl.ds(SRC_OFF // D, C2_ROWS)], cnt2_hbm)

    mbase = cid * (EP // NC) + sid * ET

    def _prep():
        @pl.loop(0, 2 * CH // 16)
        def _(i):
            ty = mtyg[pl.ds(i * 16, 16)]
            sr = msrg[pl.ds(i * 16, 16)]
            dv = mdsg[pl.ds(i * 16, 16)]
            seg = (dv - NI) * NR + ty
            cnt = plsc.load_gather(c1l, _hist_idx(seg))
            mwg[pl.ds(i * 16, 16)] = 1.0 / cnt
            mtyg[pl.ds(i * 16, 16)] = ty * NI + sr

        @pl.loop(0, CH // 16)
        def _(i):
            uix0[pl.ds(i * 16, 16)] = mdsg[pl.ds(i * 16, 16)] - NI
        @pl.loop(0, CH // 16)
        def _(i):
            uix1[pl.ds(i * 16, 16)] = mdsg[pl.ds(CH + i * 16, 16)] - NI

    def _scale(b):
        @functools.partial(plsc.parallel_loop, 0, CH, unroll=2)
        def _(e):
            w = plsc.load_gather(
                mwg, [jnp.full((16,), 1, jnp.int32) * (b * CH + e)])
            for k2 in range(8):
                val = gb[b][e, pl.ds(k2 * 16, 16)]
                gb[b][e, pl.ds(k2 * 16, 16)] = val * w

    @pl.loop(0, NCHUNK // 2)
    def _(j0):
        ebase = mbase + j0 * 2 * CH
        copies = [
            pltpu.async_copy(et_hbm.at[pl.ds(ebase, 2 * CH)], mtyg, semi),
            pltpu.async_copy(es_hbm.at[pl.ds(ebase, 2 * CH)], msrg, semi),
            pltpu.async_copy(ed_hbm.at[pl.ds(ebase, 2 * CH)], mdsg, semi),
        ]
        for c in copies:
            c.wait()
        _prep()
        g0 = pltpu.async_copy(w2_hbm.at[mtyg.at[pl.ds(0, CH)]], gb[0], semg)
        g1 = pltpu.async_copy(w2_hbm.at[mtyg.at[pl.ds(CH, CH)]], gb[1], semg)
        g0.wait()
        _scale(0)
        s0 = pltpu.async_copy(gb[0], ush.at[uix[0]], sems, add=True)
        g1.wait()
        _scale(1)
        s1 = pltpu.async_copy(gb[1], ush.at[uix[1]], sems, add=True)
        s0.wait()
        s1.wait()

    plsc.subcore_barrier()

    pltpu.sync_copy(ush.at[pl.ds(u0, CH)], gb0)
    pltpu.sync_copy(gb0, accp_hbm.at[cid, pl.ds(u0, CH)])
    pltpu.sync_copy(ush.at[pl.ds(u0 + CH, CH)], gb0)
    pltpu.sync_copy(gb0, accp_hbm.at[cid, pl.ds(u0 + CH, CH)])
    rem = UP // NS - 2 * CH
    pltpu.sync_copy(ush.at[pl.ds(u0 + 2 * CH, rem)], gb0.at[pl.ds(0, rem)])
    pltpu.sync_copy(gb0.at[pl.ds(0, rem)],
                    accp_hbm.at[cid, pl.ds(u0 + 2 * CH, rem)])


def _user_emb_body(p_ref, root_ref, bias_ref, out_ref):
    out_ref[...] = p_ref[0] + p_ref[1] + root_ref[...] + bias_ref[...]


def _make_user_emb(accp, root_u, bias_row):
    return pl.pallas_call(
        _user_emb_body,
        out_shape=jax.ShapeDtypeStruct((UP, D), jnp.float32),
    )(accp, root_u, bias_row)


@functools.partial(
    pl.kernel,
    out_type=jax.ShapeDtypeStruct((NC, IP, D), jnp.float32),
    mesh=_MESH,
    compiler_params=_SC_PARAMS,
    scratch_types=[
        pltpu.VMEM((ET,), jnp.int32),
        pltpu.VMEM((ET,), jnp.int32),
        pltpu.VMEM((CH,), jnp.int32),
        pltpu.VMEM((CH,), jnp.int32),
        pltpu.VMEM((CH,), jnp.int32),
        pltpu.VMEM((CH,), jnp.int32),
        pltpu.VMEM((CH, D), jnp.float32),
        pltpu.VMEM((CH, D), jnp.float32),
        pltpu.VMEM((CH, D), jnp.float32),
        pltpu.VMEM((CH, D), jnp.float32),
        pltpu.VMEM_SHARED((IP, D), jnp.float32),
        pltpu.SemaphoreType.DMA,
        pltpu.SemaphoreType.DMA,
        pltpu.SemaphoreType.DMA,
        pltpu.SemaphoreType.DMA,
        pltpu.SemaphoreType.DMA,
        pltpu.SemaphoreType.DMA,
        pltpu.SemaphoreType.DMA,
        pltpu.SemaphoreType.DMA,
        pltpu.SemaphoreType.DMA,
    ],
)
def _sc_stage_b(es_hbm, ed_hbm, ue_hbm, qp_hbm, ib_ds, ib_sr,
                six0, six1, six2, six3, gb0, gb1, gb2, gb3, iash,
                semg0, semg1, semg2, semg3, sems0, sems1, sems2, sems3,
                semi):
    cid = lax.axis_index("c")
    sid = lax.axis_index("s")
    six = (six0, six1, six2, six3)
    gb = (gb0, gb1, gb2, gb3)
    semg = (semg0, semg1, semg2, semg3)
    sems = (sems0, sems1, sems2, sems3)

    @pl.loop(0, CH)
    def _(i):
        for j in range(8):
            gb0[i, pl.ds(j * 16, 16)] = _zero16()

    i0 = sid * (IP // NS)
    pltpu.sync_copy(gb0, iash.at[pl.ds(i0, CH)])
    pltpu.sync_copy(gb0, iash.at[pl.ds(i0 + CH, CH)])
    plsc.subcore_barrier()

    mbase = cid * (EP // NC) + sid * ET
    cd = pltpu.async_copy(ed_hbm.at[pl.ds(mbase, ET)], ib_ds, semi)
    cs = pltpu.async_copy(es_hbm.at[pl.ds(mbase, ET)], ib_sr, semi)
    cd.wait()
    cs.wait()

    @pl.loop(0, NVEC)
    def _(i):
        ib_ds[pl.ds(i * 16, 16)] = ib_ds[pl.ds(i * 16, 16)] - NI

    def _fill(b, j):
        @pl.loop(0, CH // 16)
        def _(i):
            six[b][pl.ds(i * 16, 16)] = ib_sr[pl.ds(j * CH + i * 16, 16)]

    def _gather(b, j):
        return pltpu.async_copy(
            ue_hbm.at[ib_ds.at[pl.ds(j * CH, CH)]], gb[b], semg[b])

    for b in range(4):
        _gather(b, b)

    @pl.loop(0, NCHUNK // 4 - 1)
    def _(j0):
        for b in range(4):
            j = j0 * 4 + b
            pltpu.make_async_copy(
                ue_hbm.at[ib_ds.at[pl.ds(j * CH, CH)]], gb[b],
                semg[b]).wait()
            _fill(b, j)
            pltpu.async_copy(gb[b], iash.at[six[b]], sems[b], add=True)
        for b in range(4):
            j = j0 * 4 + b
            pltpu.make_async_copy(
                gb[b], iash.at[six[b]], sems[b]).wait()
            _gather(b, j + 4)

    for b in range(4):
        j = NCHUNK - 4 + b
        pltpu.make_async_copy(
            ue_hbm.at[ib_ds.at[pl.ds(j * CH, CH)]], gb[b], semg[b]).wait()
        _fill(b, j)
        pltpu.async_copy(gb[b], iash.at[six[b]], sems[b], add=True)
    for b in range(4):
        pltpu.make_async_copy(gb[b], iash.at[six[b]], sems[b]).wait()

    plsc.subcore_barrier()

    pltpu.sync_copy(iash.at[pl.ds(i0, CH)], gb0)
    pltpu.sync_copy(gb0, qp_hbm.at[cid, pl.ds(i0, CH)])
    pltpu.sync_copy(iash.at[pl.ds(i0 + CH, CH)], gb0)
    pltpu.sync_copy(gb0, qp_hbm.at[cid, pl.ds(i0 + CH, CH)])


def _item_mean_body(q_ref, c_ref, out_ref):
    out_ref[...] = (q_ref[0] + q_ref[1]) / jnp.maximum(c_ref[...], 1.0)


def _make_item_mean(qp, cnt_col):
    return pl.pallas_call(
        _item_mean_body,
        out_shape=jax.ShapeDtypeStruct((IP, D), jnp.float32),
    )(qp, cnt_col)


@functools.partial(
    pl.kernel,
    out_type=jax.ShapeDtypeStruct((B * L, D), jnp.float32),
    mesh=_MESH,
    compiler_params=_SC_PARAMS,
    scratch_types=[
        pltpu.VMEM((B * L // (NC * NS),), jnp.int32),
        pltpu.VMEM((CH, D), jnp.float32),
        pltpu.SemaphoreType.DMA,
    ],
)
def _sc_ctx_gather(im_hbm, ctx_hbm, h_hbm, ib, gb, sem):
    cid = lax.axis_index("c")
    sid = lax.axis_index("s")
    wid = sid * NC + cid
    per = B * L // (NC * NS)
    base = wid * per
    pltpu.sync_copy(ctx_hbm.at[pl.ds(base, per)], ib)
    for k in range(per // CH):
        pltpu.async_copy(im_hbm.at[ib.at[pl.ds(k * CH, CH)]], gb, sem).wait()
        pltpu.sync_copy(gb, h_hbm.at[pl.ds(base + k * CH, CH)])


def _attn_body(h_ref, ctx_ref, a_ref, b_ref, w1_ref, b1_ref, w2_ref, b2_ref,
               social_ref, proj_ref):
    h = h_ref[0]
    t = jnp.tanh(jnp.dot(h, a_ref[...], preferred_element_type=jnp.float32))
    e = jnp.sum(t * b_ref[...], axis=1, keepdims=True)
    valid = ctx_ref[0] >= 0
    e = jnp.where(valid, e, -1e9)
    m = jnp.max(e, axis=0, keepdims=True)
    ex = jnp.exp(e - m)
    s = jnp.sum(ex, axis=0, keepdims=True)
    alpha = ex / s
    social = h * alpha
    social_ref[0] = social
    pooled = jnp.sum(social, axis=0, keepdims=True)
    x = jnp.maximum(
        jnp.dot(pooled, w1_ref[...], preferred_element_type=jnp.float32)
        + b1_ref[...], 0.0)
    proj_ref[0] = jnp.maximum(
        jnp.dot(x, w2_ref[...], preferred_element_type=jnp.float32)
        + b2_ref[...], 0.0)


def _make_attn(h3, ctx3, attn_a, attn_b_row, fc1_w, fc1_b_row, fc2_w,
               fc2_b_row):
    return pl.pallas_call(
        _attn_body,
        grid=(B,),
        in_specs=[
            pl.BlockSpec((1, L, D), lambda i: (i, 0, 0)),
            pl.BlockSpec((1, L, 1), lambda i: (i, 0, 0)),
            pl.BlockSpec((D, D), lambda i: (0, 0)),
            pl.BlockSpec((1, D), lambda i: (0, 0)),
            pl.BlockSpec((D, D), lambda i: (0, 0)),
            pl.BlockSpec((1, D), lambda i: (0, 0)),
            pl.BlockSpec((D, D), lambda i: (0, 0)),
            pl.BlockSpec((1, D), lambda i: (0, 0)),
        ],
        out_specs=[
            pl.BlockSpec((1, L, D), lambda i: (i, 0, 0)),
            pl.BlockSpec((1, 1, D), lambda i: (i, 0, 0)),
        ],
        out_shape=[
            jax.ShapeDtypeStruct((B, L, D), jnp.float32),
            jax.ShapeDtypeStruct((B, 1, D), jnp.float32),
        ],
    )(h3, ctx3, attn_a, attn_b_row, fc1_w, fc1_b_row, fc2_w, fc2_b_row)


def kernel(context_items, edge_src, edge_dst, edge_type, basis, comp, root,
           rgcn_bias, attn_a, attn_b, fc1_w, fc1_b, fc2_w, fc2_b):
    pad = EP - E
    es = jnp.concatenate(
        [edge_src, jnp.full((pad,), PAD_SRC, jnp.int32)])
    ed = jnp.concatenate(
        [edge_dst, jnp.full((pad,), PAD_DST, jnp.int32)])
    et = jnp.concatenate(
        [edge_type, jnp.full((pad,), PAD_TY, jnp.int32)])

    w2 = _make_w2(comp, basis)

    accp, cnt2 = _sc_stage_a(es, ed, et, w2)

    root_u = jnp.concatenate(
        [root[NI:], jnp.zeros((UP - NU, D), jnp.float32)])
    user_emb = _make_user_emb(accp, root_u, rgcn_bias[None, :])

    qp = _sc_stage_b(es, ed, user_emb)

    cnt_col = cnt2.reshape(IP)[:, None]
    item_mean = _make_item_mean(qp, cnt_col)

    ctx_flat = context_items.reshape(B * L)
    h_flat = _sc_ctx_gather(item_mean, ctx_flat)

    h3 = h_flat.reshape(B, L, D)
    ctx3 = context_items.reshape(B, L, 1)
    social_reps, proj3 = _make_attn(
        h3, ctx3, attn_a, attn_b[None, :], fc1_w, fc1_b[None, :], fc2_w,
        fc2_b[None, :])
    return proj3.reshape(B, D), social_reps
